# baseline (speedup 1.0000x reference)
import functools
import jax
import jax.numpy as jnp
from jax import lax
from jax.experimental import pallas as pl
from jax.experimental.pallas import tpu as pltpu
from jax.experimental.pallas import tpu_sc as plsc

N = 10000
E = 320000
L = 16
NS = 16
NC = 2
ET = E // NS
N_PAD = 10112
ROWS_PT = N_PAD // NS

f32 = jnp.float32


def _dg(v, idx):
    return lax.gather(
        v, idx[:, None],
        dimension_numbers=lax.GatherDimensionNumbers(
            offset_dims=(), collapsed_slice_dims=(0,), start_index_map=(0,)),
        slice_sizes=(1,),
        mode=lax.GatherScatterMode.PROMISE_IN_BOUNDS)


def make_sc_edge(Fc, heads, out_dim, C=None):
    NV = Fc // L
    hpc = max(heads // NC, 1)
    if C is None:
        C = 80 if Fc >= 64 else 400
    NCHUNK = ET // C
    assert ET % C == 0 and C % 16 == 0 and NCHUNK % 2 == 0

    mesh = plsc.VectorSubcoreMesh(core_axis_name="c", subcore_axis_name="s")

    buf = lambda shape, dt=f32: pltpu.VMEM(shape, dt)

    @functools.partial(
        pl.kernel,
        out_type=[jax.ShapeDtypeStruct((NC * N_PAD, Fc), f32),
                  jax.ShapeDtypeStruct((NC * N_PAD, 16), f32)],
        mesh=mesh,
        compiler_params=pltpu.CompilerParams(use_tc_tiling_on_sc=False),
        scratch_types=[
            [buf((C,), jnp.int32), buf((C,), jnp.int32)],
            [buf((C,), jnp.int32), buf((C,), jnp.int32)],
            [buf((C,), jnp.int32), buf((C,), jnp.int32)],
            [buf((C, 16)), buf((C, 16))],
            [buf((C, 16)), buf((C, 16))],
            [buf((C, Fc)), buf((C, Fc))],
            buf((C, 16)),
            pltpu.VMEM_SHARED((N_PAD, Fc), f32),
            pltpu.VMEM_SHARED((N_PAD, 16), f32),
            [pltpu.SemaphoreType.DMA, pltpu.SemaphoreType.DMA],
            [pltpu.SemaphoreType.DMA, pltpu.SemaphoreType.DMA],
        ],
    )
    def sc_kernel(src_h, dst_h, as_h, ad_h, h_h, zf_h, z16_h,
                  agg_h, den_h,
                  sidx, didx, didx2, asv, adv, hsv, exv, agg_sh, den_sh,
                  semg, semi):
        cid = lax.axis_index("c")
        sid = lax.axis_index("s")
        lane = lax.iota(jnp.int32, 16)
        sel_perms = []
        for j in range(NV):
            if out_dim >= L:
                sel_perms.append(lane * 0 + min((j * L) // out_dim, hpc - 1))
            else:
                shift = out_dim.bit_length() - 1
                sel_perms.append(
                    lax.shift_right_logical(lane, shift) + (j * L) // out_dim)

        r0 = sid * ROWS_PT
        pltpu.sync_copy(zf_h.at[pl.ds(r0, ROWS_PT)],
                        agg_sh.at[pl.ds(r0, ROWS_PT)])
        pltpu.sync_copy(z16_h.at[pl.ds(r0, ROWS_PT)],
                        den_sh.at[pl.ds(r0, ROWS_PT)])
        plsc.subcore_barrier()

        ebase = sid * ET
        roff = cid * N

        def idx_load_async(k, p):
            base = jnp.minimum(ebase + k * C, E - C)
            c0 = pltpu.async_copy(src_h.at[pl.ds(base, C)], sidx[p], semi[p])
            c1 = pltpu.async_copy(dst_h.at[pl.ds(base, C)], didx[p], semi[p])
            return c0, c1

        def shift_and_gather(p):
            for v in range(C // L):
                sl = pl.ds(v * L, L)
                sidx[p][sl] = sidx[p][sl] + roff
                didx2[p][sl] = didx[p][sl] + roff
            g0 = pltpu.async_copy(as_h.at[sidx[p]], asv[p], semg[p])
            g1 = pltpu.async_copy(ad_h.at[didx2[p]], adv[p], semg[p])
            g2 = pltpu.async_copy(h_h.at[sidx[p]], hsv[p], semg[p])
            return g0, g1, g2

        def wait_gathers(p):
            pltpu.make_async_copy(as_h.at[sidx[p]], asv[p], semg[p]).wait()
            pltpu.make_async_copy(ad_h.at[didx2[p]], adv[p], semg[p]).wait()
            pltpu.make_async_copy(h_h.at[sidx[p]], hsv[p], semg[p]).wait()

        def compute(p):
            @plsc.parallel_loop(0, C, unroll=8)
            def edge_body(i):
                e = asv[p][i] + adv[p][i]
                e = jnp.where(e > 0.0, e, e * 0.2)
                e = jnp.minimum(e, 70.0)
                ex = jnp.exp(e)
                exv[i] = ex
                for j in range(NV):
                    m = _dg(ex, sel_perms[j])
                    hrow = hsv[p][i, pl.ds(j * L, L)]
                    hsv[p][i, pl.ds(j * L, L)] = hrow * m

        def scatter(p):
            pltpu.sync_copy(hsv[p], agg_sh.at[didx[p]], add=True)
            pltpu.sync_copy(exv, den_sh.at[didx[p]], add=True)

        i0 = idx_load_async(0, 0)
        i0[0].wait()
        i0[1].wait()
        shift_and_gather(0)

        def pair_body(m, _):
            k0 = 2 * m
            ia = idx_load_async(k0 + 1, 1)
            wait_gathers(0)
            compute(0)
            ia[0].wait()
            ia[1].wait()
            g1 = shift_and_gather(1)
            scatter(0)
            ib = idx_load_async(k0 + 2, 0)
            g1[0].wait()
            g1[1].wait()
            g1[2].wait()
            compute(1)
            ib[0].wait()
            ib[1].wait()
            shift_and_gather(0)
            scatter(1)
            return 0

        lax.fori_loop(0, NCHUNK // 2, pair_body, 0)
        wait_gathers(0)
        plsc.subcore_barrier()

        wbase = cid * N_PAD + r0
        pltpu.sync_copy(agg_sh.at[pl.ds(r0, ROWS_PT)],
                        agg_h.at[pl.ds(wbase, ROWS_PT)])
        pltpu.sync_copy(den_sh.at[pl.ds(r0, ROWS_PT)],
                        den_h.at[pl.ds(wbase, ROWS_PT)])

    return sc_kernel


@functools.lru_cache(maxsize=None)
def _get_sc(Fc, heads, out_dim):
    return make_sc_edge(Fc, heads, out_dim)


def _tc_call(body, out_shapes):
    return pl.pallas_call(
        body,
        compiler_params=pltpu.CompilerParams(
            vmem_limit_bytes=100 * 1024 * 1024),
        out_shape=[jax.ShapeDtypeStruct(s, f32) for s in out_shapes])


def tc_first(x, W, Fc):
    def body(x_ref, w_ref, h_o):
        h = jnp.dot(x_ref[...], w_ref[...], preferred_element_type=f32)
        h_o[...] = jnp.concatenate([h[:, :Fc], h[:, Fc:2 * Fc]], axis=0)

    return _tc_call(body, [(2 * N, Fc)])(x, W)[0]


_NB = 2000


def tc_alpha(Hs, Gs, Gd, hpc):
    Fc = Hs.shape[1]

    def body(h0_ref, h1_ref, gs00, gs01, gs10, gs11,
             gd00, gd01, gd10, gd11, as0_o, as1_o, ad0_o, ad1_o):
        h0 = h0_ref[...]
        h1 = h1_ref[...]
        hp = lax.Precision.HIGHEST
        lanes = lax.broadcasted_iota(jnp.int32, (1, 16), 1)
        msk = jnp.where(lanes < hpc, 0.0, -1e30).astype(f32)

        def dd(a, ga, b, gb):
            return (jnp.dot(a, ga[...], preferred_element_type=f32,
                            precision=hp)
                    + jnp.dot(b, gb[...], preferred_element_type=f32,
                              precision=hp))

        as0_o[...] = dd(h0, gs00, h1, gs01) + msk
        as1_o[...] = dd(h0, gs10, h1, gs11) + msk
        ad0_o[...] = dd(h0, gd00, h1, gd01) + msk
        ad1_o[...] = dd(h0, gd10, h1, gd11) + msk

    nb = N // _NB
    full = pl.BlockSpec((Fc, 16), lambda i: (0, 0))
    outs = pl.BlockSpec((_NB, 16), lambda i: (i, 0))
    r = pl.pallas_call(
        body,
        grid=(nb,),
        in_specs=[
            pl.BlockSpec((_NB, Fc), lambda i: (i, 0)),
            pl.BlockSpec((_NB, Fc), lambda i: (i + nb, 0)),
        ] + [full] * 8,
        out_specs=[outs] * 4,
        out_shape=[jax.ShapeDtypeStruct((N, 16), f32)] * 4,
    )(Hs, Hs, *Gs, *Gd)
    As = jnp.concatenate([r[0], r[1]], axis=0)
    Ad = jnp.concatenate([r[2], r[3]], axis=0)
    return As, Ad


def _half_norm(a, d, b, g, be, heads, out_dim):
    hpc = max(heads // NC, 1)
    if heads > 1:
        chunks = [jnp.broadcast_to(d[:, lh:lh + 1], (N, out_dim))
                  for lh in range(hpc)]
        den = jnp.concatenate(chunks, axis=1)
    else:
        den = jnp.broadcast_to(d[:, 0:1], (N, a.shape[1]))
    x = a / (den + 1e-16) + b
    x = jnp.where(x > 0.0, x, jnp.exp(x) - 1.0)
    mu = jnp.sum(x, axis=0, keepdims=True) / N
    var = jnp.sum((x - mu) * (x - mu), axis=0, keepdims=True) / N
    return (x - mu) / jnp.sqrt(var + 1e-5) * g + be


def tc_mid(agg, den, b0, b1, g0, g1, be0, be1, Wa, Wb,
           heads_p, out_p, Fc_out):
    def body(agg_ref, den_ref, b0_r, b1_r, g0_r, g1_r, be0_r, be1_r,
             wa_r, wb_r, h_o):
        xn0 = _half_norm(agg_ref[...][:N], den_ref[...][:N],
                         b0_r[...], g0_r[...], be0_r[...], heads_p, out_p)
        xn1 = _half_norm(agg_ref[...][N_PAD:N_PAD + N],
                         den_ref[...][N_PAD:N_PAD + N],
                         b1_r[...], g1_r[...], be1_r[...], heads_p, out_p)
        h = (jnp.dot(xn0, wa_r[...], preferred_element_type=f32)
             + jnp.dot(xn1, wb_r[...], preferred_element_type=f32))
        h_o[...] = jnp.concatenate(
            [h[:, :Fc_out], h[:, Fc_out:2 * Fc_out]], axis=0)

    return _tc_call(body, [(2 * N, Fc_out)])(
        agg, den, b0, b1, g0, g1, be0, be1, Wa, Wb)[0]


def tc_last(agg, den, b, g, be):
    def body(agg_ref, den_ref, b_ref, g_ref, be_ref, o_ref):
        a0 = agg_ref[...][:N]
        a1 = agg_ref[...][N_PAD:N_PAD + N][:, :8]
        d0 = den_ref[...][:N]
        d1 = den_ref[...][N_PAD:N_PAD + N]
        den0 = jnp.broadcast_to(d0[:, 0:1], (N, 32))
        den1 = jnp.broadcast_to(d1[:, 0:1], (N, 8))
        agg_c = jnp.concatenate([a0, a1], axis=1)
        den_c = jnp.concatenate([den0, den1], axis=1)
        x = agg_c / (den_c + 1e-16) + b_ref[...]
        mu = jnp.mean(x, axis=0, keepdims=True)
        var = jnp.mean((x - mu) * (x - mu), axis=0, keepdims=True)
        o_ref[...] = (x - mu) / jnp.sqrt(var + 1e-5) * g_ref[...] + be_ref[...]

    return _tc_call(body, [(N, 40)])(agg, den, b, g, be)[0]


def _build_T(avec, core):
    heads, out_dim = avec.shape
    hpc = max(heads // NC, 1)
    hid = jnp.repeat(jnp.arange(heads), out_dim)
    lh = hid - (core * hpc if heads > 1 else 0)
    valid = (lh >= 0) & (lh < hpc)
    lh = jnp.clip(lh, 0, hpc - 1)
    eye = jnp.eye(16, dtype=f32)
    v = valid.astype(f32)[:, None]
    return (eye[lh] * avec.reshape(-1)[:, None] * v).astype(f32)


def _split_G(Gfull, Fc):
    Frows = Gfull.shape[0]
    if Frows < 2 * Fc:
        Gfull = jnp.concatenate(
            [Gfull, jnp.zeros((2 * Fc - Frows, 16), f32)], axis=0)
    return Gfull[:Fc], Gfull[Fc:2 * Fc]


def kernel(x, edge_index, W1, as1, ad1, b1, g1, be1, W2, as2, ad2, b2, g2, be2,
           W3, as3, ad3, b3, g3, be3, W4, as4, ad4, b4, g4, be4):
    src = edge_index[0]
    dst = edge_index[1]
    Gs = {}
    Gd = {}
    for i, (a_s, a_d, Fc) in enumerate(
            [(as1, ad1, 128), (as2, ad2, 64), (as3, ad3, 32),
             (as4, ad4, 32)], start=1):
        Gs[i] = _split_G(_build_T(a_s, 0), Fc) + _split_G(_build_T(a_s, 1), Fc)
        Gd[i] = _split_G(_build_T(a_d, 0), Fc) + _split_G(_build_T(a_d, 1), Fc)
    z128 = jnp.zeros((N_PAD, 128), f32)
    z64 = jnp.zeros((N_PAD, 64), f32)
    z32 = jnp.zeros((N_PAD, 32), f32)
    z16 = jnp.zeros((N_PAD, 16), f32)
    W4p = jnp.concatenate([W4, jnp.zeros((64, 24), f32)], axis=1)

    def halves(v):
        m = v.shape[0] // 2
        return v[:m].reshape(1, -1), v[m:].reshape(1, -1)

    H = tc_first(x, W1, 128)
    As, Ad = tc_alpha(H, Gs[1], Gd[1], 4)
    agg, den = _get_sc(128, 8, 32)(src, dst, As, Ad, H, z128, z16)

    b0, b1h = halves(b1)
    g0, g1h = halves(g1)
    be0, be1h = halves(be1)
    H = tc_mid(agg, den, b0, b1h, g0, g1h, be0, be1h, W2[:128], W2[128:],
               8, 32, 64)
    As, Ad = tc_alpha(H, Gs[2], Gd[2], 4)
    agg, den = _get_sc(64, 8, 16)(src, dst, As, Ad, H, z64, z16)

    b0, b1h = halves(b2)
    g0, g1h = halves(g2)
    be0, be1h = halves(be2)
    H = tc_mid(agg, den, b0, b1h, g0, g1h, be0, be1h, W3[:64], W3[64:],
               8, 16, 32)
    As, Ad = tc_alpha(H, Gs[3], Gd[3], 4)
    agg, den = _get_sc(32, 8, 8)(src, dst, As, Ad, H, z32, z16)

    b0, b1h = halves(b3)
    g0, g1h = halves(g3)
    be0, be1h = halves(be3)
    H = tc_mid(agg, den, b0, b1h, g0, g1h, be0, be1h, W4p[:32], W4p[32:],
               8, 8, 32)
    As, Ad = tc_alpha(H, Gs[4], Gd[4], 1)
    agg, den = _get_sc(32, 1, 40)(src, dst, As, Ad, H, z32, z16)

    out = tc_last(agg, den, b4.reshape(1, -1), g4.reshape(1, -1),
                  be4.reshape(1, -1))
    return out


# speedup vs baseline: 1.0516x; 1.0516x over previous
import functools
import jax
import jax.numpy as jnp
from jax import lax
from jax.experimental import pallas as pl
from jax.experimental.pallas import tpu as pltpu
from jax.experimental.pallas import tpu_sc as plsc

N = 10000
E = 320000
L = 16
NS = 16
NC = 2
ET = E // NS
N_PAD = 10112
ROWS_PT = N_PAD // NS

f32 = jnp.float32


def _dg(v, idx):
    return lax.gather(
        v, idx[:, None],
        dimension_numbers=lax.GatherDimensionNumbers(
            offset_dims=(), collapsed_slice_dims=(0,), start_index_map=(0,)),
        slice_sizes=(1,),
        mode=lax.GatherScatterMode.PROMISE_IN_BOUNDS)


def make_sc_edge(Fc, heads, out_dim, C=None):
    NV = Fc // L
    hpc = max(heads // NC, 1)
    if C is None:
        C = 80 if Fc >= 128 else (160 if Fc >= 64 else 400)
    NCHUNK = ET // C
    assert ET % C == 0 and C % 16 == 0

    mesh = plsc.VectorSubcoreMesh(core_axis_name="c", subcore_axis_name="s")

    buf = lambda shape, dt=f32: pltpu.VMEM(shape, dt)

    @functools.partial(
        pl.kernel,
        out_type=[jax.ShapeDtypeStruct((NC * N_PAD, Fc), f32),
                  jax.ShapeDtypeStruct((NC * N_PAD, 16), f32)],
        mesh=mesh,
        compiler_params=pltpu.CompilerParams(use_tc_tiling_on_sc=False),
        scratch_types=[
            [buf((C,), jnp.int32), buf((C,), jnp.int32)],
            [buf((C,), jnp.int32), buf((C,), jnp.int32)],
            [buf((C,), jnp.int32), buf((C,), jnp.int32)],
            [buf((C, 16)), buf((C, 16))],
            [buf((C, 16)), buf((C, 16))],
            [buf((C, Fc)), buf((C, Fc))],
            buf((C, 16)),
            pltpu.VMEM_SHARED((N_PAD, Fc), f32),
            pltpu.VMEM_SHARED((N_PAD, 16), f32),
            [pltpu.SemaphoreType.DMA, pltpu.SemaphoreType.DMA],
            [pltpu.SemaphoreType.DMA, pltpu.SemaphoreType.DMA],
        ],
    )
    def sc_kernel(src_h, dst_h, as_h, ad_h, h_h, zf_h, z16_h,
                  agg_h, den_h,
                  sidx, didx, didx2, asv, adv, hsv, exv, agg_sh, den_sh,
                  semg, semi):
        cid = lax.axis_index("c")
        sid = lax.axis_index("s")
        lane = lax.iota(jnp.int32, 16)
        sel_perms = []
        for j in range(NV):
            if out_dim >= L:
                sel_perms.append(lane * 0 + min((j * L) // out_dim, hpc - 1))
            else:
                shift = out_dim.bit_length() - 1
                sel_perms.append(
                    lax.shift_right_logical(lane, shift) + (j * L) // out_dim)

        r0 = sid * ROWS_PT
        pltpu.sync_copy(zf_h.at[pl.ds(r0, ROWS_PT)],
                        agg_sh.at[pl.ds(r0, ROWS_PT)])
        pltpu.sync_copy(z16_h.at[pl.ds(r0, ROWS_PT)],
                        den_sh.at[pl.ds(r0, ROWS_PT)])
        plsc.subcore_barrier()

        ebase = sid * ET
        roff = cid * N

        def idx_load_async(k, p):
            base = jnp.minimum(ebase + k * C, E - C)
            c0 = pltpu.async_copy(src_h.at[pl.ds(base, C)], sidx[p], semi[p])
            c1 = pltpu.async_copy(dst_h.at[pl.ds(base, C)], didx[p], semi[p])
            return c0, c1

        def shift_and_gather(p):
            for v in range(C // L):
                sl = pl.ds(v * L, L)
                sidx[p][sl] = sidx[p][sl] + roff
                didx2[p][sl] = didx[p][sl] + roff
            g0 = pltpu.async_copy(as_h.at[sidx[p]], asv[p], semg[p])
            g1 = pltpu.async_copy(ad_h.at[didx2[p]], adv[p], semg[p])
            g2 = pltpu.async_copy(h_h.at[sidx[p]], hsv[p], semg[p])
            return g0, g1, g2

        def wait_gathers(p):
            pltpu.make_async_copy(as_h.at[sidx[p]], asv[p], semg[p]).wait()
            pltpu.make_async_copy(ad_h.at[didx2[p]], adv[p], semg[p]).wait()
            pltpu.make_async_copy(h_h.at[sidx[p]], hsv[p], semg[p]).wait()

        def compute(p):
            @plsc.parallel_loop(0, C, unroll=4)
            def edge_body(i):
                e = asv[p][i] + adv[p][i]
                e = jnp.where(e > 0.0, e, e * 0.2)
                e = jnp.minimum(e, 70.0)
                ex = jnp.exp(e)
                exv[i] = ex
                for j in range(NV):
                    m = _dg(ex, sel_perms[j])
                    hrow = hsv[p][i, pl.ds(j * L, L)]
                    hsv[p][i, pl.ds(j * L, L)] = hrow * m

        def scatter(p):
            pltpu.sync_copy(hsv[p], agg_sh.at[didx[p]], add=True)
            pltpu.sync_copy(exv, den_sh.at[didx[p]], add=True)

        i0 = idx_load_async(0, 0)
        i0[0].wait()
        i0[1].wait()
        shift_and_gather(0)

        def pair_body(m, _):
            k0 = 2 * m
            ia = idx_load_async(k0 + 1, 1)
            wait_gathers(0)
            compute(0)
            ia[0].wait()
            ia[1].wait()
            g1 = shift_and_gather(1)
            scatter(0)
            ib = idx_load_async(k0 + 2, 0)
            g1[0].wait()
            g1[1].wait()
            g1[2].wait()
            compute(1)
            ib[0].wait()
            ib[1].wait()
            shift_and_gather(0)
            scatter(1)
            return 0

        lax.fori_loop(0, NCHUNK // 2, pair_body, 0)
        if NCHUNK % 2 == 1:
            wait_gathers(0)
            compute(0)
            scatter(0)
        else:
            wait_gathers(0)
        plsc.subcore_barrier()

        wbase = cid * N_PAD + r0
        pltpu.sync_copy(agg_sh.at[pl.ds(r0, ROWS_PT)],
                        agg_h.at[pl.ds(wbase, ROWS_PT)])
        pltpu.sync_copy(den_sh.at[pl.ds(r0, ROWS_PT)],
                        den_h.at[pl.ds(wbase, ROWS_PT)])

    return sc_kernel


@functools.lru_cache(maxsize=None)
def _get_sc(Fc, heads, out_dim):
    return make_sc_edge(Fc, heads, out_dim)


def _tc_call(body, out_shapes):
    return pl.pallas_call(
        body,
        compiler_params=pltpu.CompilerParams(
            vmem_limit_bytes=100 * 1024 * 1024),
        out_shape=[jax.ShapeDtypeStruct(s, f32) for s in out_shapes])


def tc_first(x, W, Fc):
    def body(x_ref, w_ref, h_o):
        h = jnp.dot(x_ref[...], w_ref[...], preferred_element_type=f32)
        h_o[...] = jnp.concatenate([h[:, :Fc], h[:, Fc:2 * Fc]], axis=0)

    return _tc_call(body, [(2 * N, Fc)])(x, W)[0]


_NB = 2000


def tc_alpha(Hs, Gs, Gd, hpc):
    Fc = Hs.shape[1]

    def body(h0_ref, h1_ref, gs0, gs1, gs2, gs3, gd0, gd1, gd2, gd3,
             as0_o, as1_o, ad0_o, ad1_o):
        h0 = h0_ref[...]
        h1 = h1_ref[...]
        hp = lax.Precision.HIGHEST
        lanes = lax.broadcasted_iota(jnp.int32, (1, 16), 1)
        msk = jnp.where(lanes < hpc, 0.0, -1e30).astype(f32)

        def dd(ga, gb):
            return (jnp.dot(h0, ga[...], preferred_element_type=f32,
                            precision=hp)
                    + jnp.dot(h1, gb[...], preferred_element_type=f32,
                              precision=hp))

        as0_o[...] = dd(gs0, gs1) + msk
        as1_o[...] = dd(gs2, gs3) + msk
        ad0_o[...] = dd(gd0, gd1) + msk
        ad1_o[...] = dd(gd2, gd3) + msk

    nb = N // _NB
    full = pl.BlockSpec((Fc, 16), lambda i: (0, 0))
    outs = pl.BlockSpec((_NB, 16), lambda i: (i, 0))
    r = pl.pallas_call(
        body,
        grid=(nb,),
        in_specs=[
            pl.BlockSpec((_NB, Fc), lambda i: (i, 0)),
            pl.BlockSpec((_NB, Fc), lambda i: (i + nb, 0)),
        ] + [full] * 8,
        out_specs=[outs] * 4,
        out_shape=[jax.ShapeDtypeStruct((N, 16), f32)] * 4,
    )(Hs, Hs, *Gs, *Gd)
    As = jnp.concatenate([r[0], r[1]], axis=0)
    Ad = jnp.concatenate([r[2], r[3]], axis=0)
    return As, Ad


def _half_norm(a, d, b, g, be, heads, out_dim):
    hpc = max(heads // NC, 1)
    if heads > 1:
        chunks = [jnp.broadcast_to(d[:, lh:lh + 1], (N, out_dim))
                  for lh in range(hpc)]
        den = jnp.concatenate(chunks, axis=1)
    else:
        den = jnp.broadcast_to(d[:, 0:1], (N, a.shape[1]))
    x = a / (den + 1e-16) + b
    x = jnp.where(x > 0.0, x, jnp.exp(x) - 1.0)
    mu = jnp.sum(x, axis=0, keepdims=True) / N
    var = jnp.sum((x - mu) * (x - mu), axis=0, keepdims=True) / N
    return (x - mu) / jnp.sqrt(var + 1e-5) * g + be


def tc_mid(agg, den, b0, b1, g0, g1, be0, be1, Wa, Wb,
           heads_p, out_p, Fc_out):
    def body(agg_ref, den_ref, b0_r, b1_r, g0_r, g1_r, be0_r, be1_r,
             wa_r, wb_r, h_o):
        xn0 = _half_norm(agg_ref[...][:N], den_ref[...][:N],
                         b0_r[...], g0_r[...], be0_r[...], heads_p, out_p)
        xn1 = _half_norm(agg_ref[...][N_PAD:N_PAD + N],
                         den_ref[...][N_PAD:N_PAD + N],
                         b1_r[...], g1_r[...], be1_r[...], heads_p, out_p)
        h = (jnp.dot(xn0, wa_r[...], preferred_element_type=f32)
             + jnp.dot(xn1, wb_r[...], preferred_element_type=f32))
        h_o[...] = jnp.concatenate(
            [h[:, :Fc_out], h[:, Fc_out:2 * Fc_out]], axis=0)

    return _tc_call(body, [(2 * N, Fc_out)])(
        agg, den, b0, b1, g0, g1, be0, be1, Wa, Wb)[0]


def tc_last(agg, den, b, g, be):
    def body(agg_ref, den_ref, b_ref, g_ref, be_ref, o_ref):
        a0 = agg_ref[...][:N]
        a1 = agg_ref[...][N_PAD:N_PAD + N][:, :8]
        d0 = den_ref[...][:N]
        d1 = den_ref[...][N_PAD:N_PAD + N]
        den0 = jnp.broadcast_to(d0[:, 0:1], (N, 32))
        den1 = jnp.broadcast_to(d1[:, 0:1], (N, 8))
        agg_c = jnp.concatenate([a0, a1], axis=1)
        den_c = jnp.concatenate([den0, den1], axis=1)
        x = agg_c / (den_c + 1e-16) + b_ref[...]
        mu = jnp.mean(x, axis=0, keepdims=True)
        var = jnp.mean((x - mu) * (x - mu), axis=0, keepdims=True)
        o_ref[...] = (x - mu) / jnp.sqrt(var + 1e-5) * g_ref[...] + be_ref[...]

    return _tc_call(body, [(N, 40)])(agg, den, b, g, be)[0]


def _build_T(avec, core):
    heads, out_dim = avec.shape
    hpc = max(heads // NC, 1)
    hid = jnp.repeat(jnp.arange(heads), out_dim)
    lh = hid - (core * hpc if heads > 1 else 0)
    valid = (lh >= 0) & (lh < hpc)
    lh = jnp.clip(lh, 0, hpc - 1)
    eye = jnp.eye(16, dtype=f32)
    v = valid.astype(f32)[:, None]
    return (eye[lh] * avec.reshape(-1)[:, None] * v).astype(f32)


def _split_G(Gfull, Fc):
    Frows = Gfull.shape[0]
    if Frows < 2 * Fc:
        Gfull = jnp.concatenate(
            [Gfull, jnp.zeros((2 * Fc - Frows, 16), f32)], axis=0)
    return Gfull[:Fc], Gfull[Fc:2 * Fc]


def kernel(x, edge_index, W1, as1, ad1, b1, g1, be1, W2, as2, ad2, b2, g2, be2,
           W3, as3, ad3, b3, g3, be3, W4, as4, ad4, b4, g4, be4):
    src = edge_index[0]
    dst = edge_index[1]
    Gs = {}
    Gd = {}
    for i, (a_s, a_d, Fc) in enumerate(
            [(as1, ad1, 128), (as2, ad2, 64), (as3, ad3, 32),
             (as4, ad4, 32)], start=1):
        Gs[i] = _split_G(_build_T(a_s, 0), Fc) + _split_G(_build_T(a_s, 1), Fc)
        Gd[i] = _split_G(_build_T(a_d, 0), Fc) + _split_G(_build_T(a_d, 1), Fc)
    z128 = jnp.zeros((N_PAD, 128), f32)
    z64 = jnp.zeros((N_PAD, 64), f32)
    z32 = jnp.zeros((N_PAD, 32), f32)
    z16 = jnp.zeros((N_PAD, 16), f32)
    W4p = jnp.concatenate([W4, jnp.zeros((64, 24), f32)], axis=1)

    def halves(v):
        m = v.shape[0] // 2
        return v[:m].reshape(1, -1), v[m:].reshape(1, -1)

    H = tc_first(x, W1, 128)
    As, Ad = tc_alpha(H, Gs[1], Gd[1], 4)
    agg, den = _get_sc(128, 8, 32)(src, dst, As, Ad, H, z128, z16)

    b0, b1h = halves(b1)
    g0, g1h = halves(g1)
    be0, be1h = halves(be1)
    H = tc_mid(agg, den, b0, b1h, g0, g1h, be0, be1h,
               W2[:128], W2[128:], 8, 32, 64)
    As, Ad = tc_alpha(H, Gs[2], Gd[2], 4)
    agg, den = _get_sc(64, 8, 16)(src, dst, As, Ad, H, z64, z16)

    b0, b1h = halves(b2)
    g0, g1h = halves(g2)
    be0, be1h = halves(be2)
    H = tc_mid(agg, den, b0, b1h, g0, g1h, be0, be1h,
               W3[:64], W3[64:], 8, 16, 32)
    As, Ad = tc_alpha(H, Gs[3], Gd[3], 4)
    agg, den = _get_sc(32, 8, 8)(src, dst, As, Ad, H, z32, z16)

    b0, b1h = halves(b3)
    g0, g1h = halves(g3)
    be0, be1h = halves(be3)
    H = tc_mid(agg, den, b0, b1h, g0, g1h, be0, be1h,
               W4p[:32], W4p[32:], 8, 8, 32)
    As, Ad = tc_alpha(H, Gs[4], Gd[4], 1)
    agg, den = _get_sc(32, 1, 40)(src, dst, As, Ad, H, z32, z16)

    out = tc_last(agg, den, b4.reshape(1, -1), g4.reshape(1, -1),
                  be4.reshape(1, -1))
    return out
